# Initial kernel scaffold; baseline (speedup 1.0000x reference)
import jax
import jax.numpy as jnp
from jax.experimental import pallas as pl


def kernel(x, edge_index, edge_type, batch, shape_emb, color_emb, W1_rel, W1_root, b1, W2_rel, W2_root, b2, Wc, bc):
    raise NotImplementedError("write your pallas kernel here")


# speedup vs baseline: 1.4629x; 1.4629x over previous
import functools

import jax
import jax.numpy as jnp
from jax import lax
from jax.experimental import pallas as pl
from jax.experimental.pallas import tpu as pltpu
from jax.experimental.pallas import tpu_sc as plsc

N_NODES = 100000
N_EDGES = 1600000
EMB = 16
HID = 32
NUM_REL = 3
N_GRAPHS = 128
N_CLASSES = 10

NC = 2
NS = 16

CHUNK = 16768
PASSES = 6
ACC_ROWS = 3 * CHUNK
TRASH = ACC_ROWS
ACC_ALLOC = ACC_ROWS + 8
OUT_ROWS = PASSES * ACC_ROWS
STRIPE = ACC_ROWS // NS
W = 2048
E_SUB = 102400
E_PAD = NS * E_SUB
N_WIN = E_SUB // W
ZROWS = STRIPE // 3
assert 3 * ZROWS == STRIPE and STRIPE % 8 == 0 and ZROWS % 8 == 0


def _sc_agg(h_lo, h_hi, src, dst, et, with_cnt):
  out_type = [
      jax.ShapeDtypeStruct((OUT_ROWS, 16), jnp.float32),
      jax.ShapeDtypeStruct((OUT_ROWS, 16), jnp.float32),
  ]
  if with_cnt:
    out_type.append(jax.ShapeDtypeStruct((OUT_ROWS,), jnp.float32))

  mesh = plsc.VectorSubcoreMesh(core_axis_name="c", subcore_axis_name="s")

  def body(h_lo_hbm, h_hi_hbm, src_hbm, dst_hbm, et_hbm, *rest):
    if with_cnt:
      agg_lo_hbm, agg_hi_hbm, cnt_hbm = rest[:3]
      scratch = rest[3:]
    else:
      agg_lo_hbm, agg_hi_hbm = rest[:2]
      cnt_hbm = None
      scratch = rest[2:]
    (src_v, dst_v, et_v, idx2_v, rows_v, ones_v, zb1_v, zrows_v,
     acc_sh, cnt_sh, sem) = scratch

    c = lax.axis_index("c")
    s = lax.axis_index("s")

    zvec = jnp.zeros((16,), jnp.float32)
    ovec = jnp.ones((16,), jnp.float32)

    def fill_zrows(r, _):
      zrows_v[r, :] = zvec
      return 0
    lax.fori_loop(0, ZROWS, fill_zrows, 0)

    def fill_vecs(i, _):
      zb1_v[pl.ds(i * 16, 16)] = zvec
      ones_v[pl.ds(i * 16, 16)] = ovec
      return 0
    lax.fori_loop(0, W // 16, fill_vecs, 0)

    def run_half(h_hbm, agg_hbm, do_cnt):
      for p in range(PASSES):
        lo = p * CHUNK
        hi = lo + CHUNK
        for z in range(3):
          pltpu.sync_copy(
              zrows_v, acc_sh.at[pl.ds(s * STRIPE + z * ZROWS, ZROWS)])
        if do_cnt:
          for z in range(3):
            pltpu.sync_copy(
                zb1_v.at[pl.ds(0, ZROWS)],
                cnt_sh.at[pl.ds(s * STRIPE + z * ZROWS, ZROWS)])
        plsc.subcore_barrier()

        def window(w, _):
          eoff = s * E_SUB + w * W
          pltpu.sync_copy(src_hbm.at[pl.ds(eoff, W)], src_v)
          pltpu.sync_copy(dst_hbm.at[pl.ds(eoff, W)], dst_v)
          pltpu.sync_copy(et_hbm.at[pl.ds(eoff, W)], et_v)

          def compute_idx(k, _):
            for jj in range(8):
              e0 = (k * 8 + jj) * 16
              d = dst_v[pl.ds(e0, 16)]
              t = et_v[pl.ds(e0, 16)]
              inr = (d >= lo) & (d < hi)
              comb = jnp.where(inr, (d - lo) * 3 + t, TRASH)
              idx2_v[k, pl.ds(jj * 16, 16)] = comb
            return 0
          lax.fori_loop(0, 16, compute_idx, 0)

          pltpu.async_copy(h_hbm.at[src_v], rows_v, sem).wait()

          for k in range(16):
            pltpu.sync_copy(rows_v.at[pl.ds(k * 128, 128)],
                            acc_sh.at[idx2_v.at[k]], add=True)
            if do_cnt:
              pltpu.sync_copy(ones_v.at[pl.ds(k * 128, 128)],
                              cnt_sh.at[idx2_v.at[k]], add=True)
          return 0

        lax.fori_loop(0, N_WIN, window, 0)
        plsc.subcore_barrier()

        pltpu.sync_copy(acc_sh.at[pl.ds(s * STRIPE, STRIPE)],
                        agg_hbm.at[pl.ds(p * ACC_ROWS + s * STRIPE, STRIPE)])
        if do_cnt:
          pltpu.sync_copy(cnt_sh.at[pl.ds(s * STRIPE, STRIPE)],
                          cnt_hbm.at[pl.ds(p * ACC_ROWS + s * STRIPE, STRIPE)])
        plsc.subcore_barrier()

    @pl.when(c == 0)
    def _():
      run_half(h_lo_hbm, agg_lo_hbm, with_cnt)

    @pl.when(c == 1)
    def _():
      run_half(h_hi_hbm, agg_hi_hbm, False)

  scratch_types = [
      pltpu.VMEM((W,), jnp.int32),
      pltpu.VMEM((W,), jnp.int32),
      pltpu.VMEM((W,), jnp.int32),
      pltpu.VMEM((16, 128), jnp.int32),
      pltpu.VMEM((W, 16), jnp.float32),
      pltpu.VMEM((W,), jnp.float32),
      pltpu.VMEM((W,), jnp.float32),
      pltpu.VMEM((ZROWS, 16), jnp.float32),
      pltpu.VMEM_SHARED((ACC_ALLOC, 16), jnp.float32),
      pltpu.VMEM_SHARED((ACC_ALLOC,), jnp.float32),
      pltpu.SemaphoreType.DMA,
  ]

  k = pl.kernel(body, out_type=tuple(out_type), mesh=mesh,
                scratch_types=scratch_types,
                compiler_params=pltpu.CompilerParams(
                    use_tc_tiling_on_sc=False))
  return k(h_lo, h_hi, src, dst, et)


def _embed_kernel(x_ref, se_ref, ce_ref, lo_ref, hi_ref):
  b = x_ref.shape[0]
  i16 = lax.broadcasted_iota(jnp.int32, (b, 16), 1)
  oh_s = (x_ref[:, 0:1] == i16).astype(jnp.float32)
  oh_c = (x_ref[:, 1:2] == i16).astype(jnp.float32)
  lo_ref[...] = jnp.dot(oh_s, se_ref[...], preferred_element_type=jnp.float32)
  hi_ref[...] = jnp.dot(oh_c, ce_ref[...], preferred_element_type=jnp.float32)


def _embed(x, shape_emb, color_emb):
  B = 512
  grid = (N_NODES + B - 1) // B
  return pl.pallas_call(
      _embed_kernel,
      grid=(grid,),
      in_specs=[
          pl.BlockSpec((B, 2), lambda i: (i, 0)),
          pl.BlockSpec((16, EMB), lambda i: (0, 0)),
          pl.BlockSpec((16, EMB), lambda i: (0, 0)),
      ],
      out_specs=[
          pl.BlockSpec((B, 16), lambda i: (i, 0)),
          pl.BlockSpec((B, 16), lambda i: (i, 0)),
      ],
      out_shape=[
          jax.ShapeDtypeStruct((N_NODES, 16), jnp.float32),
          jax.ShapeDtypeStruct((N_NODES, 16), jnp.float32),
      ],
  )(x, shape_emb, color_emb)


def _combine_kernel(hlo_ref, hhi_ref, alo_ref, ahi_ref, cnt_ref, w_ref, b_ref,
                    olo_ref, ohi_ref):
  inv = 1.0 / jnp.maximum(cnt_ref[...], 1.0)
  alo = alo_ref[...] * inv[:, :, None]
  ahi = ahi_ref[...] * inv[:, :, None]
  feat = jnp.concatenate(
      [hlo_ref[...], hhi_ref[...],
       alo[:, 0, :], ahi[:, 0, :],
       alo[:, 1, :], ahi[:, 1, :],
       alo[:, 2, :], ahi[:, 2, :]], axis=1)
  out = jnp.dot(feat, w_ref[...], preferred_element_type=jnp.float32)
  out = jnp.maximum(out + b_ref[...], 0.0)
  olo_ref[...] = out[:, :16]
  ohi_ref[...] = out[:, 16:]


def _combine(h_lo, h_hi, agg_lo, agg_hi, cnt, w_root, w_rel, bias):
  wcat = jnp.concatenate([w_root, w_rel[0], w_rel[1], w_rel[2]], axis=0)
  b2d = bias.reshape(1, HID)
  agg_lo3 = agg_lo[:3 * N_NODES].reshape(N_NODES, 3, 16)
  agg_hi3 = agg_hi[:3 * N_NODES].reshape(N_NODES, 3, 16)
  cnt3 = cnt[:3 * N_NODES].reshape(N_NODES, 3)
  B = 512
  grid = (N_NODES + B - 1) // B
  return pl.pallas_call(
      _combine_kernel,
      grid=(grid,),
      in_specs=[
          pl.BlockSpec((B, 16), lambda i: (i, 0)),
          pl.BlockSpec((B, 16), lambda i: (i, 0)),
          pl.BlockSpec((B, 3, 16), lambda i: (i, 0, 0)),
          pl.BlockSpec((B, 3, 16), lambda i: (i, 0, 0)),
          pl.BlockSpec((B, 3), lambda i: (i, 0)),
          pl.BlockSpec((2 * EMB * 4, HID), lambda i: (0, 0)),
          pl.BlockSpec((1, HID), lambda i: (0, 0)),
      ],
      out_specs=[
          pl.BlockSpec((B, 16), lambda i: (i, 0)),
          pl.BlockSpec((B, 16), lambda i: (i, 0)),
      ],
      out_shape=[
          jax.ShapeDtypeStruct((N_NODES, 16), jnp.float32),
          jax.ShapeDtypeStruct((N_NODES, 16), jnp.float32),
      ],
  )(h_lo, h_hi, agg_lo3, agg_hi3, cnt3, wcat, b2d)


def _pool_kernel(hlo_ref, hhi_ref, batch_ref, wc_ref, bc_ref, out_ref, acc):
  i = pl.program_id(0)
  n_i = pl.num_programs(0)
  b = hlo_ref.shape[0]

  @pl.when(i == 0)
  def _():
    acc[...] = jnp.zeros_like(acc)

  rowid = i * b + lax.broadcasted_iota(jnp.int32, (b, 1), 0)
  valid = rowid < N_NODES
  x = jnp.concatenate(
      [hlo_ref[...], hhi_ref[...], jnp.ones((b, 1), jnp.float32)], axis=1)
  x = jnp.where(valid, x, 0.0)
  oh = (batch_ref[...] == lax.broadcasted_iota(jnp.int32, (b, N_GRAPHS), 1))
  part = lax.dot_general(oh.astype(jnp.float32), x,
                         (((0,), (0,)), ((), ())),
                         preferred_element_type=jnp.float32)
  acc[...] = acc[...] + part

  @pl.when(i == n_i - 1)
  def _():
    z = acc[...]
    g = z[:, :HID] / jnp.maximum(z[:, HID:HID + 1], 1.0)
    out_ref[...] = (
        jnp.dot(g, wc_ref[...], preferred_element_type=jnp.float32)
        + bc_ref[...])


def _pool_classify(h_lo, h_hi, batch, wc, bc):
  B = 512
  grid = (N_NODES + B - 1) // B
  return pl.pallas_call(
      _pool_kernel,
      grid=(grid,),
      in_specs=[
          pl.BlockSpec((B, 16), lambda i: (i, 0)),
          pl.BlockSpec((B, 16), lambda i: (i, 0)),
          pl.BlockSpec((B, 1), lambda i: (i, 0)),
          pl.BlockSpec((HID, N_CLASSES), lambda i: (0, 0)),
          pl.BlockSpec((1, N_CLASSES), lambda i: (0, 0)),
      ],
      out_specs=pl.BlockSpec((N_GRAPHS, N_CLASSES), lambda i: (0, 0)),
      out_shape=jax.ShapeDtypeStruct((N_GRAPHS, N_CLASSES), jnp.float32),
      scratch_shapes=[pltpu.VMEM((N_GRAPHS, HID + 1), jnp.float32)],
  )(h_lo, h_hi, batch.reshape(N_NODES, 1), wc, bc.reshape(1, N_CLASSES))


def kernel(x, edge_index, edge_type, batch, shape_emb, color_emb,
           W1_rel, W1_root, b1, W2_rel, W2_root, b2, Wc, bc):
  x = x.astype(jnp.int32)
  src = edge_index[0].astype(jnp.int32)
  dst = edge_index[1].astype(jnp.int32)
  et = edge_type.astype(jnp.int32)
  batch = batch.astype(jnp.int32)

  pad = E_PAD - N_EDGES
  src_p = jnp.concatenate([src, jnp.zeros((pad,), jnp.int32)])
  dst_p = jnp.concatenate([dst, jnp.full((pad,), N_NODES, jnp.int32)])
  et_p = jnp.concatenate([et, jnp.zeros((pad,), jnp.int32)])

  h0_lo, h0_hi = _embed(x, shape_emb, color_emb)
  agg1_lo, agg1_hi, cnt = _sc_agg(h0_lo, h0_hi, src_p, dst_p, et_p, True)
  h1_lo, h1_hi = _combine(h0_lo, h0_hi, agg1_lo, agg1_hi, cnt,
                          W1_root, W1_rel, b1)
  agg2_lo, agg2_hi = _sc_agg(h1_lo, h1_hi, src_p, dst_p, et_p, False)
  h2_lo, h2_hi = _combine(h1_lo, h1_hi, agg2_lo, agg2_hi, cnt,
                          W2_root, W2_rel, b2)
  return _pool_classify(h2_lo, h2_hi, batch, Wc, bc)


# speedup vs baseline: 4.3856x; 2.9979x over previous
import functools

import jax
import jax.numpy as jnp
from jax import lax
from jax.experimental import pallas as pl
from jax.experimental.pallas import tpu as pltpu
from jax.experimental.pallas import tpu_sc as plsc

N_NODES = 100000
N_EDGES = 1600000
EMB = 16
HID = 32
NUM_REL = 3
N_GRAPHS = 128
N_CLASSES = 10

NC = 2
NS = 16

CHUNK = 16768
PASSES = 6
ACC_ROWS = 3 * CHUNK
TRASH = ACC_ROWS
ACC_ALLOC = ACC_ROWS + 8
OUT_ROWS = PASSES * ACC_ROWS
STRIPE = ACC_ROWS // NS
SPAN = 6400
N_SPAN = 16
E_SUB = SPAN * N_SPAN
E_PAD = NS * E_SUB
CH = 1024
DUMP = SPAN + CH
SEL_CAP = DUMP + 16
CSEL_ROWS = 59
ZROWS = STRIPE // 3
assert 3 * ZROWS == STRIPE and STRIPE % 8 == 0 and ZROWS % 8 == 0


def _sc_agg(h_lo, h_hi, src, dst, et, with_cnt):
  out_type = [
      jax.ShapeDtypeStruct((OUT_ROWS, 16), jnp.float32),
      jax.ShapeDtypeStruct((OUT_ROWS, 16), jnp.float32),
  ]
  if with_cnt:
    out_type.append(jax.ShapeDtypeStruct((OUT_ROWS,), jnp.float32))

  mesh = plsc.VectorSubcoreMesh(core_axis_name="c", subcore_axis_name="s")

  def body(h_lo_hbm, h_hi_hbm, src_hbm, dst_hbm, et_hbm, *rest):
    if with_cnt:
      agg_lo_hbm, agg_hi_hbm, cnt_hbm = rest[:3]
      scratch = rest[3:]
    else:
      agg_lo_hbm, agg_hi_hbm = rest[:2]
      cnt_hbm = None
      scratch = rest[2:]
    (src_v, dst_v, et_v, ssel_v, csel2_v, rows_v,
     ones_v, zb1_v, zrows_v, acc_sh, cnt_sh, sem) = scratch

    c = lax.axis_index("c")
    s = lax.axis_index("s")

    zvec = jnp.zeros((16,), jnp.float32)
    ovec = jnp.ones((16,), jnp.float32)
    iota16 = lax.iota(jnp.int32, 16)
    padsrc = s * 16 + iota16
    padcomb = jnp.full((16,), TRASH, jnp.int32)

    def fill_zrows(r, _):
      zrows_v[r, :] = zvec
      return 0
    lax.fori_loop(0, ZROWS, fill_zrows, 0)

    def fill_vecs(i, _):
      zb1_v[pl.ds(i * 16, 16)] = zvec
      ones_v[pl.ds(i * 16, 16)] = ovec
      return 0
    lax.fori_loop(0, CH // 16, fill_vecs, 0)

    def run_half(h_hbm, agg_hbm, do_cnt):
      for p in range(PASSES):
        lo = p * CHUNK
        hi = lo + CHUNK
        for z in range(3):
          pltpu.sync_copy(
              zrows_v, acc_sh.at[pl.ds(s * STRIPE + z * ZROWS, ZROWS)])
        if do_cnt:
          for z in range(3):
            pltpu.sync_copy(
                zb1_v.at[pl.ds(0, ZROWS)],
                cnt_sh.at[pl.ds(s * STRIPE + z * ZROWS, ZROWS)])
        plsc.subcore_barrier()

        def span_body(sp, _):
          eoff = s * E_SUB + sp * SPAN
          pltpu.sync_copy(src_hbm.at[pl.ds(eoff, SPAN)], src_v)
          pltpu.sync_copy(dst_hbm.at[pl.ds(eoff, SPAN)], dst_v)
          pltpu.sync_copy(et_hbm.at[pl.ds(eoff, SPAN)], et_v)

          def grp(j, off_vec):
            d = dst_v[pl.ds(j * 16, 16)]
            t = et_v[pl.ds(j * 16, 16)]
            sg = src_v[pl.ds(j * 16, 16)]
            inr = (d >= lo) & (d < hi)
            comb = (d - lo) * 3 + t
            mi = jnp.where(inr, 1, 0)
            csum = plsc.cumsum(mi)
            pc = plsc.all_reduce_population_count(inr)
            pos = jnp.where(inr, off_vec + csum - 1, DUMP + iota16)
            plsc.store_scatter(ssel_v, [pos], sg)
            plsc.store_scatter(csel2_v, [pos >> 7, pos & 127], comb)
            return off_vec + pc
          off_vec = lax.fori_loop(0, SPAN // 16, grp,
                                  jnp.zeros((16,), jnp.int32))
          total = off_vec[0]

          nch = (total + CH - 1) // CH

          def padgrp(k, _):
            base = total + k * 16
            ssel_v[pl.ds(base, 16)] = padsrc
            plsc.store_scatter(csel2_v, [(base + iota16) >> 7,
                                         (base + iota16) & 127], padcomb)
            return 0
          lax.fori_loop(0, CH // 16, padgrp, 0)

          def chunk_body(ci, _):
            pltpu.async_copy(
                h_hbm.at[ssel_v.at[pl.ds(ci * CH, CH)]], rows_v, sem).wait()
            for k in range(CH // 128):
              row = csel2_v.at[ci * (CH // 128) + k]
              pltpu.sync_copy(rows_v.at[pl.ds(k * 128, 128)],
                              acc_sh.at[row], add=True)
              if do_cnt:
                pltpu.sync_copy(ones_v.at[pl.ds(k * 128, 128)],
                                cnt_sh.at[row], add=True)
            return 0
          lax.fori_loop(0, nch, chunk_body, 0)
          return 0

        lax.fori_loop(0, N_SPAN, span_body, 0)
        plsc.subcore_barrier()

        pltpu.sync_copy(acc_sh.at[pl.ds(s * STRIPE, STRIPE)],
                        agg_hbm.at[pl.ds(p * ACC_ROWS + s * STRIPE, STRIPE)])
        if do_cnt:
          pltpu.sync_copy(cnt_sh.at[pl.ds(s * STRIPE, STRIPE)],
                          cnt_hbm.at[pl.ds(p * ACC_ROWS + s * STRIPE, STRIPE)])
        plsc.subcore_barrier()

    @pl.when(c == 0)
    def _():
      run_half(h_lo_hbm, agg_lo_hbm, with_cnt)

    @pl.when(c == 1)
    def _():
      run_half(h_hi_hbm, agg_hi_hbm, False)

  scratch_types = [
      pltpu.VMEM((SPAN,), jnp.int32),
      pltpu.VMEM((SPAN,), jnp.int32),
      pltpu.VMEM((SPAN,), jnp.int32),
      pltpu.VMEM((SEL_CAP,), jnp.int32),
      pltpu.VMEM((CSEL_ROWS, 128), jnp.int32),
      pltpu.VMEM((CH, 16), jnp.float32),
      pltpu.VMEM((CH,), jnp.float32),
      pltpu.VMEM((CH,), jnp.float32),
      pltpu.VMEM((ZROWS, 16), jnp.float32),
      pltpu.VMEM_SHARED((ACC_ALLOC, 16), jnp.float32),
      pltpu.VMEM_SHARED((ACC_ALLOC,), jnp.float32),
      pltpu.SemaphoreType.DMA,
  ]

  k = pl.kernel(body, out_type=tuple(out_type), mesh=mesh,
                scratch_types=scratch_types,
                compiler_params=pltpu.CompilerParams(
                    use_tc_tiling_on_sc=False,
                    needs_layout_passes=False))
  return k(h_lo, h_hi, src, dst, et)


def _embed_kernel(x_ref, se_ref, ce_ref, lo_ref, hi_ref):
  b = x_ref.shape[0]
  i16 = lax.broadcasted_iota(jnp.int32, (b, 16), 1)
  oh_s = (x_ref[:, 0:1] == i16).astype(jnp.float32)
  oh_c = (x_ref[:, 1:2] == i16).astype(jnp.float32)
  lo_ref[...] = jnp.dot(oh_s, se_ref[...], preferred_element_type=jnp.float32)
  hi_ref[...] = jnp.dot(oh_c, ce_ref[...], preferred_element_type=jnp.float32)


def _embed(x, shape_emb, color_emb):
  B = 512
  grid = (N_NODES + B - 1) // B
  return pl.pallas_call(
      _embed_kernel,
      grid=(grid,),
      in_specs=[
          pl.BlockSpec((B, 2), lambda i: (i, 0)),
          pl.BlockSpec((16, EMB), lambda i: (0, 0)),
          pl.BlockSpec((16, EMB), lambda i: (0, 0)),
      ],
      out_specs=[
          pl.BlockSpec((B, 16), lambda i: (i, 0)),
          pl.BlockSpec((B, 16), lambda i: (i, 0)),
      ],
      out_shape=[
          jax.ShapeDtypeStruct((N_NODES, 16), jnp.float32),
          jax.ShapeDtypeStruct((N_NODES, 16), jnp.float32),
      ],
  )(x, shape_emb, color_emb)


def _combine_kernel(hlo_ref, hhi_ref, alo_ref, ahi_ref, cnt_ref, w_ref, b_ref,
                    olo_ref, ohi_ref):
  inv = 1.0 / jnp.maximum(cnt_ref[...], 1.0)
  alo = alo_ref[...] * inv[:, :, None]
  ahi = ahi_ref[...] * inv[:, :, None]
  feat = jnp.concatenate(
      [hlo_ref[...], hhi_ref[...],
       alo[:, 0, :], ahi[:, 0, :],
       alo[:, 1, :], ahi[:, 1, :],
       alo[:, 2, :], ahi[:, 2, :]], axis=1)
  out = jnp.dot(feat, w_ref[...], preferred_element_type=jnp.float32)
  out = jnp.maximum(out + b_ref[...], 0.0)
  olo_ref[...] = out[:, :16]
  ohi_ref[...] = out[:, 16:]


def _combine(h_lo, h_hi, agg_lo, agg_hi, cnt, w_root, w_rel, bias):
  wcat = jnp.concatenate([w_root, w_rel[0], w_rel[1], w_rel[2]], axis=0)
  b2d = bias.reshape(1, HID)
  agg_lo3 = agg_lo[:3 * N_NODES].reshape(N_NODES, 3, 16)
  agg_hi3 = agg_hi[:3 * N_NODES].reshape(N_NODES, 3, 16)
  cnt3 = cnt[:3 * N_NODES].reshape(N_NODES, 3)
  B = 512
  grid = (N_NODES + B - 1) // B
  return pl.pallas_call(
      _combine_kernel,
      grid=(grid,),
      in_specs=[
          pl.BlockSpec((B, 16), lambda i: (i, 0)),
          pl.BlockSpec((B, 16), lambda i: (i, 0)),
          pl.BlockSpec((B, 3, 16), lambda i: (i, 0, 0)),
          pl.BlockSpec((B, 3, 16), lambda i: (i, 0, 0)),
          pl.BlockSpec((B, 3), lambda i: (i, 0)),
          pl.BlockSpec((2 * EMB * 4, HID), lambda i: (0, 0)),
          pl.BlockSpec((1, HID), lambda i: (0, 0)),
      ],
      out_specs=[
          pl.BlockSpec((B, 16), lambda i: (i, 0)),
          pl.BlockSpec((B, 16), lambda i: (i, 0)),
      ],
      out_shape=[
          jax.ShapeDtypeStruct((N_NODES, 16), jnp.float32),
          jax.ShapeDtypeStruct((N_NODES, 16), jnp.float32),
      ],
  )(h_lo, h_hi, agg_lo3, agg_hi3, cnt3, wcat, b2d)


def _pool_kernel(hlo_ref, hhi_ref, batch_ref, wc_ref, bc_ref, out_ref, acc):
  i = pl.program_id(0)
  n_i = pl.num_programs(0)
  b = hlo_ref.shape[0]

  @pl.when(i == 0)
  def _():
    acc[...] = jnp.zeros_like(acc)

  rowid = i * b + lax.broadcasted_iota(jnp.int32, (b, 1), 0)
  valid = rowid < N_NODES
  x = jnp.concatenate(
      [hlo_ref[...], hhi_ref[...], jnp.ones((b, 1), jnp.float32)], axis=1)
  x = jnp.where(valid, x, 0.0)
  oh = (batch_ref[...] == lax.broadcasted_iota(jnp.int32, (b, N_GRAPHS), 1))
  part = lax.dot_general(oh.astype(jnp.float32), x,
                         (((0,), (0,)), ((), ())),
                         preferred_element_type=jnp.float32)
  acc[...] = acc[...] + part

  @pl.when(i == n_i - 1)
  def _():
    z = acc[...]
    g = z[:, :HID] / jnp.maximum(z[:, HID:HID + 1], 1.0)
    out_ref[...] = (
        jnp.dot(g, wc_ref[...], preferred_element_type=jnp.float32)
        + bc_ref[...])


def _pool_classify(h_lo, h_hi, batch, wc, bc):
  B = 512
  grid = (N_NODES + B - 1) // B
  return pl.pallas_call(
      _pool_kernel,
      grid=(grid,),
      in_specs=[
          pl.BlockSpec((B, 16), lambda i: (i, 0)),
          pl.BlockSpec((B, 16), lambda i: (i, 0)),
          pl.BlockSpec((B, 1), lambda i: (i, 0)),
          pl.BlockSpec((HID, N_CLASSES), lambda i: (0, 0)),
          pl.BlockSpec((1, N_CLASSES), lambda i: (0, 0)),
      ],
      out_specs=pl.BlockSpec((N_GRAPHS, N_CLASSES), lambda i: (0, 0)),
      out_shape=jax.ShapeDtypeStruct((N_GRAPHS, N_CLASSES), jnp.float32),
      scratch_shapes=[pltpu.VMEM((N_GRAPHS, HID + 1), jnp.float32)],
  )(h_lo, h_hi, batch.reshape(N_NODES, 1), wc, bc.reshape(1, N_CLASSES))


def kernel(x, edge_index, edge_type, batch, shape_emb, color_emb,
           W1_rel, W1_root, b1, W2_rel, W2_root, b2, Wc, bc):
  x = x.astype(jnp.int32)
  src = edge_index[0].astype(jnp.int32)
  dst = edge_index[1].astype(jnp.int32)
  et = edge_type.astype(jnp.int32)
  batch = batch.astype(jnp.int32)

  pad = E_PAD - N_EDGES
  src_p = jnp.concatenate([src, jnp.zeros((pad,), jnp.int32)])
  dst_p = jnp.concatenate([dst, jnp.full((pad,), N_NODES, jnp.int32)])
  et_p = jnp.concatenate([et, jnp.zeros((pad,), jnp.int32)])

  h0_lo, h0_hi = _embed(x, shape_emb, color_emb)
  agg1_lo, agg1_hi, cnt = _sc_agg(h0_lo, h0_hi, src_p, dst_p, et_p, True)
  h1_lo, h1_hi = _combine(h0_lo, h0_hi, agg1_lo, agg1_hi, cnt,
                          W1_root, W1_rel, b1)
  agg2_lo, agg2_hi = _sc_agg(h1_lo, h1_hi, src_p, dst_p, et_p, False)
  h2_lo, h2_hi = _combine(h1_lo, h1_hi, agg2_lo, agg2_hi, cnt,
                          W2_root, W2_rel, b2)
  return _pool_classify(h2_lo, h2_hi, batch, Wc, bc)
